# baseline (speedup 1.0000x reference)
import functools

import jax
import jax.numpy as jnp
from jax import lax
from jax.experimental import pallas as pl
from jax.experimental.pallas import tpu as pltpu
from jax.experimental.pallas import tpu_sc as plsc

N = 10000
E = 160000
H = 256
HH = 128
NTILE = 16
BATCH = 64
NB = 160
EPAD = NTILE * NB * BATCH
DUMP = N
NPAD = 10112
ROWS = NPAD
TROWS = NPAD // NTILE


def _dot_t(a, w):
    return lax.dot_general(a, w, (((1,), (1,)), ((), ())),
                           preferred_element_type=jnp.float32)


_BM = 2000


def _encoder_body(x_ref, w_ref, b_ref, emb_ref, out_ref):
    acc = _dot_t(x_ref[...], w_ref[...]) + b_ref[0] + emb_ref[...]
    out_ref[...] = acc[None]


def _tc_encoder(x_material, W_lin, b_lin, emb_material):
    b2 = b_lin.reshape(2, 1, HH)
    return pl.pallas_call(
        _encoder_body,
        grid=(2, N // _BM),
        in_specs=[
            pl.BlockSpec((_BM, H), lambda c, r: (r, 0)),
            pl.BlockSpec((HH, H), lambda c, r: (c, 0)),
            pl.BlockSpec((1, 1, HH), lambda c, r: (c, 0, 0)),
            pl.BlockSpec((_BM, HH), lambda c, r: (r, c)),
        ],
        out_specs=pl.BlockSpec((1, _BM, HH), lambda c, r: (c, r, 0)),
        out_shape=jax.ShapeDtypeStruct((2, N, HH), jnp.float32),
    )(x_material, W_lin, b2, emb_material)


def _combine_body(aggr_ref, cnt_ref, xd_ref, wl_ref, wr_ref, b_ref, out_ref,
                  *, lrelu):
    inv = 1.0 / jnp.maximum(cnt_ref[:, 0:1], 1.0)
    a0 = aggr_ref[0] * inv
    a1 = aggr_ref[1] * inv
    acc = (_dot_t(a0, wl_ref[:, 0:HH]) + _dot_t(a1, wl_ref[:, HH:H])
           + _dot_t(xd_ref[0], wr_ref[:, 0:HH]) + _dot_t(xd_ref[1], wr_ref[:, HH:H])
           + b_ref[0])
    if lrelu:
        acc = jnp.where(acc > 0, acc, 0.01 * acc)
    out_ref[...] = acc[None]


def _tc_combine(aggr, cnt, x_dst, W_l, b_l, W_r, lrelu):
    b2 = b_l.reshape(2, 1, HH)
    return pl.pallas_call(
        functools.partial(_combine_body, lrelu=lrelu),
        grid=(2, N // _BM),
        in_specs=[
            pl.BlockSpec((2, _BM, HH), lambda c, r: (0, r, 0)),
            pl.BlockSpec((_BM, HH), lambda c, r: (r, 0)),
            pl.BlockSpec((2, _BM, HH), lambda c, r: (0, r, 0)),
            pl.BlockSpec((HH, H), lambda c, r: (c, 0)),
            pl.BlockSpec((HH, H), lambda c, r: (c, 0)),
            pl.BlockSpec((1, 1, HH), lambda c, r: (c, 0, 0)),
        ],
        out_specs=pl.BlockSpec((1, _BM, HH), lambda c, r: (c, r, 0)),
        out_shape=jax.ShapeDtypeStruct((2, N, HH), jnp.float32),
    )(aggr, cnt, x_dst, W_l, W_r, b2)


def _head_body(x_ref, w_ref, b_ref, out_ref):
    out_ref[...] = (_dot_t(x_ref[0], w_ref[:, 0:HH])
                    + _dot_t(x_ref[1], w_ref[:, HH:H]) + b_ref[...])


def _tc_head(x, W_out, b_out):
    nout = W_out.shape[0]
    return pl.pallas_call(
        _head_body,
        grid=(N // _BM,),
        in_specs=[
            pl.BlockSpec((2, _BM, HH), lambda r: (0, r, 0)),
            pl.BlockSpec((nout, H), lambda r: (0, 0)),
            pl.BlockSpec((1, nout), lambda r: (0, 0)),
        ],
        out_specs=pl.BlockSpec((_BM, nout), lambda r: (r, 0)),
        out_shape=jax.ShapeDtypeStruct((N, nout), jnp.float32),
    )(x, W_out, b_out.reshape(1, nout))


_MESH = plsc.VectorSubcoreMesh(core_axis_name="c", subcore_axis_name="s")


def _fill(ref, rows, value):
    vec = jnp.full((16,), value, jnp.float32)

    def body(i, _):
        for j in range(ref.shape[1] // 16):
            ref[i, pl.ds(j * 16, 16)] = vec
        return 0

    lax.fori_loop(0, rows, body, 0)


def _zero_slab(zsrc, acc_sh, slab):
    nfull = TROWS // BATCH
    for k in range(nfull):
        pltpu.sync_copy(zsrc, acc_sh.at[pl.ds(slab + k * BATCH, BATCH)])
    rem = TROWS - nfull * BATCH
    if rem:
        pltpu.sync_copy(zsrc.at[pl.ds(0, rem)],
                        acc_sh.at[pl.ds(slab + nfull * BATCH, rem)])


@functools.partial(
    pl.kernel,
    out_type=jax.ShapeDtypeStruct((2, NPAD, HH), jnp.float32),
    mesh=_MESH,
    scratch_types=[
        pltpu.VMEM((NB // 4, BATCH), jnp.int32),
        pltpu.VMEM((NB // 4, BATCH), jnp.int32),
        pltpu.VMEM((4, BATCH, HH), jnp.float32),
        [pltpu.SemaphoreType.DMA] * 4,
        [pltpu.SemaphoreType.DMA] * 4,
        pltpu.VMEM_SHARED((ROWS, HH), jnp.float32),
    ],
)
def _sc_segsum(x_hbm, srcs_hbm, dsts_hbm, out_hbm,
               src_v, dst_v, rows_v, gsems, ssems, acc_sh):
    c = lax.axis_index("c")
    s = lax.axis_index("s")
    PH = NB // 4
    bufs = tuple(rows_v.at[b] for b in range(4))
    _fill(bufs[0], BATCH, 0.0)
    slab = s * TROWS
    _zero_slab(bufs[0], acc_sh, slab)
    plsc.subcore_barrier()

    def wait_gather(b):
        pltpu.make_async_copy(x_hbm.at[src_v.at[0]], bufs[b], gsems[b]).wait()

    def wait_scatter(b):
        pltpu.make_async_copy(bufs[b], acc_sh.at[dst_v.at[0]], ssems[b]).wait()

    for p in range(4):
        pltpu.sync_copy(srcs_hbm.at[c, s, pl.ds(p * PH, PH)], src_v)
        pltpu.sync_copy(dsts_hbm.at[s, pl.ds(p * PH, PH)], dst_v)
        pltpu.async_copy(x_hbm.at[src_v.at[0]], bufs[0], gsems[0])
        pltpu.async_copy(x_hbm.at[src_v.at[1]], bufs[1], gsems[1])

        def body(i, _):
            for b in range(4):
                j = 4 * i + b
                b2 = (b + 2) % 4
                wait_gather(b)
                pltpu.async_copy(bufs[b], acc_sh.at[dst_v.at[j]], ssems[b],
                                 add=True)

                @pl.when(j >= 2)
                def _():
                    wait_scatter(b2)

                @pl.when(j + 2 < PH)
                def _():
                    pltpu.async_copy(x_hbm.at[src_v.at[j + 2]], bufs[b2],
                                     gsems[b2])
            return 0

        lax.fori_loop(0, PH // 4, body, 0)
        wait_scatter((PH - 2) % 4)
        wait_scatter((PH - 1) % 4)
    plsc.subcore_barrier()
    pltpu.sync_copy(acc_sh.at[pl.ds(slab, TROWS)],
                    out_hbm.at[c, pl.ds(slab, TROWS)])


@functools.partial(
    pl.kernel,
    out_type=jax.ShapeDtypeStruct((2, NPAD, HH), jnp.float32),
    mesh=_MESH,
    scratch_types=[
        pltpu.VMEM((NB, BATCH), jnp.int32),
        pltpu.VMEM((BATCH, HH), jnp.float32),
        pltpu.VMEM_SHARED((ROWS, HH), jnp.float32),
    ],
)
def _sc_counts(dsts_hbm, out_hbm, dst_v, ones_v, acc_sh):
    c = lax.axis_index("c")
    s = lax.axis_index("s")
    pltpu.sync_copy(dsts_hbm.at[c, s], dst_v)
    _fill(ones_v, BATCH, 0.0)
    slab = s * TROWS
    _zero_slab(ones_v, acc_sh, slab)
    plsc.subcore_barrier()
    _fill(ones_v, BATCH, 1.0)

    def body(j, _):
        pltpu.sync_copy(ones_v, acc_sh.at[dst_v.at[j]], add=True)
        return 0

    lax.fori_loop(0, NB, body, 0)
    plsc.subcore_barrier()
    pltpu.sync_copy(acc_sh.at[pl.ds(slab, TROWS)],
                    out_hbm.at[c, pl.ds(slab, TROWS)])


def _prep_edges(ei):
    src = ei[0].astype(jnp.int32)
    dst = ei[1].astype(jnp.int32)
    order = jnp.argsort(src)
    src = src[order]
    dst = dst[order]
    pad = EPAD - E
    src = jnp.concatenate([src, jnp.zeros((pad,), jnp.int32)])
    dst = jnp.concatenate([dst, jnp.full((pad,), DUMP, jnp.int32)])
    src = src.reshape(NTILE, NB, BATCH)
    dst = dst.reshape(NTILE, NB, BATCH)
    srcs = jnp.stack([src, src + N])
    return srcs, dst


def kernel(x_material, emb_material, emb_element, W_lin, b_lin, W1_me_l,
           b1_me_l, W1_me_r, W1_em_l, b1_em_l, W1_em_r, W2_me_l, b2_me_l,
           W2_me_r, W2_em_l, b2_em_l, W2_em_r, W_out, b_out,
           node_id_material, node_id_element, ei_me, ei_em):
    srcs_me, dst_me = _prep_edges(ei_me)
    srcs_em, dst_em = _prep_edges(ei_em)

    cnts = _sc_counts(jnp.stack([dst_me, dst_em]))
    cnt_e = cnts[0]
    cnt_m = cnts[1]

    x_mat0 = _tc_encoder(x_material, W_lin, b_lin, emb_material)
    x_elem0 = emb_element.reshape(N, 2, HH).transpose(1, 0, 2)

    agg_e1 = _sc_segsum(x_mat0.reshape(2 * N, HH), srcs_me, dst_me)
    agg_m1 = _sc_segsum(x_elem0.reshape(2 * N, HH), srcs_em, dst_em)
    x_elem1 = _tc_combine(agg_e1, cnt_e, x_elem0, W1_me_l, b1_me_l, W1_me_r,
                          lrelu=True)
    x_mat1 = _tc_combine(agg_m1, cnt_m, x_mat0, W1_em_l, b1_em_l, W1_em_r,
                         lrelu=True)

    agg_m2 = _sc_segsum(x_elem1.reshape(2 * N, HH), srcs_em, dst_em)
    x_mat2 = _tc_combine(agg_m2, cnt_m, x_mat1, W2_em_l, b2_em_l, W2_em_r,
                         lrelu=True)

    return _tc_head(x_mat2, W_out, b_out)


# speedup vs baseline: 1.5798x; 1.5798x over previous
import functools

import jax
import jax.numpy as jnp
from jax import lax
from jax.experimental import pallas as pl
from jax.experimental.pallas import tpu as pltpu
from jax.experimental.pallas import tpu_sc as plsc

N = 10000
E = 160000
H = 256
HH = 128
NTILE = 16
BATCH = 64
NB = 160
EPAD = NTILE * NB * BATCH
DUMP = N
NPAD = 10112
ROWS = NPAD
TROWS = NPAD // NTILE


def _dot_t(a, w):
    return lax.dot_general(a, w, (((1,), (1,)), ((), ())),
                           preferred_element_type=jnp.float32)


_BM = 2000


def _encoder_body(x_ref, w_ref, b_ref, emb_ref, out_ref):
    acc = _dot_t(x_ref[...], w_ref[...]) + b_ref[0] + emb_ref[...]
    out_ref[...] = acc[None]


def _tc_encoder(x_material, W_lin, b_lin, emb_material):
    b2 = b_lin.reshape(2, 1, HH)
    return pl.pallas_call(
        _encoder_body,
        grid=(2, N // _BM),
        in_specs=[
            pl.BlockSpec((_BM, H), lambda c, r: (r, 0)),
            pl.BlockSpec((HH, H), lambda c, r: (c, 0)),
            pl.BlockSpec((1, 1, HH), lambda c, r: (c, 0, 0)),
            pl.BlockSpec((_BM, HH), lambda c, r: (r, c)),
        ],
        out_specs=pl.BlockSpec((1, _BM, HH), lambda c, r: (c, r, 0)),
        out_shape=jax.ShapeDtypeStruct((2, N, HH), jnp.float32),
    )(x_material, W_lin, b2, emb_material)


def _combine_body(aggr_ref, cnt_ref, xd_ref, wl_ref, wr_ref, b_ref, out_ref,
                  *, lrelu):
    inv = 1.0 / jnp.maximum(cnt_ref[:, 0:1], 1.0)
    a0 = aggr_ref[0] * inv
    a1 = aggr_ref[1] * inv
    acc = (_dot_t(a0, wl_ref[:, 0:HH]) + _dot_t(a1, wl_ref[:, HH:H])
           + _dot_t(xd_ref[0], wr_ref[:, 0:HH]) + _dot_t(xd_ref[1], wr_ref[:, HH:H])
           + b_ref[0])
    if lrelu:
        acc = jnp.where(acc > 0, acc, 0.01 * acc)
    out_ref[...] = acc[None]


def _tc_combine(aggr, cnt, x_dst, W_l, b_l, W_r, lrelu):
    b2 = b_l.reshape(2, 1, HH)
    return pl.pallas_call(
        functools.partial(_combine_body, lrelu=lrelu),
        grid=(2, N // _BM),
        in_specs=[
            pl.BlockSpec((2, _BM, HH), lambda c, r: (0, r, 0)),
            pl.BlockSpec((_BM, HH), lambda c, r: (r, 0)),
            pl.BlockSpec((2, _BM, HH), lambda c, r: (0, r, 0)),
            pl.BlockSpec((HH, H), lambda c, r: (c, 0)),
            pl.BlockSpec((HH, H), lambda c, r: (c, 0)),
            pl.BlockSpec((1, 1, HH), lambda c, r: (c, 0, 0)),
        ],
        out_specs=pl.BlockSpec((1, _BM, HH), lambda c, r: (c, r, 0)),
        out_shape=jax.ShapeDtypeStruct((2, N, HH), jnp.float32),
    )(aggr, cnt, x_dst, W_l, W_r, b2)


def _head_body(x_ref, w_ref, b_ref, out_ref):
    out_ref[...] = (_dot_t(x_ref[0], w_ref[:, 0:HH])
                    + _dot_t(x_ref[1], w_ref[:, HH:H]) + b_ref[...])


def _tc_head(x, W_out, b_out):
    nout = W_out.shape[0]
    return pl.pallas_call(
        _head_body,
        grid=(N // _BM,),
        in_specs=[
            pl.BlockSpec((2, _BM, HH), lambda r: (0, r, 0)),
            pl.BlockSpec((nout, H), lambda r: (0, 0)),
            pl.BlockSpec((1, nout), lambda r: (0, 0)),
        ],
        out_specs=pl.BlockSpec((_BM, nout), lambda r: (r, 0)),
        out_shape=jax.ShapeDtypeStruct((N, nout), jnp.float32),
    )(x, W_out, b_out.reshape(1, nout))


_MESH = plsc.VectorSubcoreMesh(core_axis_name="c", subcore_axis_name="s")


def _fill(ref, rows, value):
    vec = jnp.full((16,), value, jnp.float32)

    def body(i, _):
        for j in range(ref.shape[1] // 16):
            ref[i, pl.ds(j * 16, 16)] = vec
        return 0

    lax.fori_loop(0, rows, body, 0)


def _zero_slab(zsrc, acc_sh, slab):
    nfull = TROWS // BATCH
    for k in range(nfull):
        pltpu.sync_copy(zsrc, acc_sh.at[pl.ds(slab + k * BATCH, BATCH)])
    rem = TROWS - nfull * BATCH
    if rem:
        pltpu.sync_copy(zsrc.at[pl.ds(0, rem)],
                        acc_sh.at[pl.ds(slab + nfull * BATCH, rem)])


def _make_segsum(W):
  @functools.partial(
    pl.kernel,
    out_type=jax.ShapeDtypeStruct((2, NPAD, W), jnp.float32),
    mesh=_MESH,
    scratch_types=[
        pltpu.VMEM((NB // 4, BATCH), jnp.int32),
        pltpu.VMEM((NB // 4, BATCH), jnp.int32),
        pltpu.VMEM((4, BATCH, W), jnp.float32),
        [pltpu.SemaphoreType.DMA] * 4,
        [pltpu.SemaphoreType.DMA] * 4,
        pltpu.VMEM_SHARED((ROWS, W), jnp.float32),
    ],
  )
  def _sc_segsum(x_hbm, srcs_hbm, dsts_hbm, out_hbm,
                 src_v, dst_v, rows_v, gsems, ssems, acc_sh):
      c = lax.axis_index("c")
      s = lax.axis_index("s")
      PH = NB // 4
      bufs = tuple(rows_v.at[b] for b in range(4))
      _fill(bufs[0], BATCH, 0.0)
      slab = s * TROWS
      _zero_slab(bufs[0], acc_sh, slab)
      plsc.subcore_barrier()

      def wait_gather(b):
          pltpu.make_async_copy(x_hbm.at[src_v.at[0]], bufs[b], gsems[b]).wait()

      def wait_scatter(b):
          pltpu.make_async_copy(bufs[b], acc_sh.at[dst_v.at[0]], ssems[b]).wait()

      for p in range(4):
          pltpu.sync_copy(srcs_hbm.at[c, s, pl.ds(p * PH, PH)], src_v)
          pltpu.sync_copy(dsts_hbm.at[s, pl.ds(p * PH, PH)], dst_v)
          pltpu.async_copy(x_hbm.at[src_v.at[0]], bufs[0], gsems[0])
          pltpu.async_copy(x_hbm.at[src_v.at[1]], bufs[1], gsems[1])

          def body(i, _):
              for b in range(4):
                  j = 4 * i + b
                  b2 = (b + 2) % 4
                  wait_gather(b)
                  pltpu.async_copy(bufs[b], acc_sh.at[dst_v.at[j]], ssems[b],
                                   add=True)

                  @pl.when(j >= 2)
                  def _():
                      wait_scatter(b2)

                  @pl.when(j + 2 < PH)
                  def _():
                      pltpu.async_copy(x_hbm.at[src_v.at[j + 2]], bufs[b2],
                                       gsems[b2])
              return 0

          lax.fori_loop(0, PH // 4, body, 0)
          wait_scatter((PH - 2) % 4)
          wait_scatter((PH - 1) % 4)
      plsc.subcore_barrier()
      pltpu.sync_copy(acc_sh.at[pl.ds(slab, TROWS)],
                      out_hbm.at[c, pl.ds(slab, TROWS)])
  return _sc_segsum


_sc_segsum = _make_segsum(HH)


@functools.partial(
    pl.kernel,
    out_type=jax.ShapeDtypeStruct((2, NPAD, HH), jnp.float32),
    mesh=_MESH,
    scratch_types=[
        pltpu.VMEM((NB, BATCH), jnp.int32),
        pltpu.VMEM((BATCH, HH), jnp.float32),
        pltpu.VMEM_SHARED((ROWS, HH), jnp.float32),
    ],
)
def _sc_counts(dsts_hbm, out_hbm, dst_v, ones_v, acc_sh):
    c = lax.axis_index("c")
    s = lax.axis_index("s")
    pltpu.sync_copy(dsts_hbm.at[c, s], dst_v)
    _fill(ones_v, BATCH, 0.0)
    slab = s * TROWS
    _zero_slab(ones_v, acc_sh, slab)
    plsc.subcore_barrier()
    _fill(ones_v, BATCH, 1.0)

    def body(j, _):
        pltpu.sync_copy(ones_v, acc_sh.at[dst_v.at[j]], add=True)
        return 0

    lax.fori_loop(0, NB, body, 0)
    plsc.subcore_barrier()
    pltpu.sync_copy(acc_sh.at[pl.ds(slab, TROWS)],
                    out_hbm.at[c, pl.ds(slab, TROWS)])


def _prep_edges(ei):
    src = ei[0].astype(jnp.int32)
    dst = ei[1].astype(jnp.int32)
    pad = EPAD - E
    src = jnp.concatenate([src, jnp.zeros((pad,), jnp.int32)])
    dst = jnp.concatenate([dst, jnp.full((pad,), DUMP, jnp.int32)])
    src = src.reshape(NTILE, NB, BATCH)
    dst = dst.reshape(NTILE, NB, BATCH)
    srcs = jnp.stack([src, src + N])
    return srcs, dst


def kernel(x_material, emb_material, emb_element, W_lin, b_lin, W1_me_l,
           b1_me_l, W1_me_r, W1_em_l, b1_em_l, W1_em_r, W2_me_l, b2_me_l,
           W2_me_r, W2_em_l, b2_em_l, W2_em_r, W_out, b_out,
           node_id_material, node_id_element, ei_me, ei_em):
    srcs_me, dst_me = _prep_edges(ei_me)
    srcs_em, dst_em = _prep_edges(ei_em)

    cnts = _sc_counts(jnp.stack([dst_me, dst_em]))
    cnt_e = cnts[0]
    cnt_m = cnts[1]

    x_mat0 = _tc_encoder(x_material, W_lin, b_lin, emb_material)
    x_elem0 = emb_element.reshape(N, 2, HH).transpose(1, 0, 2)

    agg_e1 = _sc_segsum(x_mat0.reshape(2 * N, HH), srcs_me, dst_me)
    agg_m1 = _sc_segsum(x_elem0.reshape(2 * N, HH), srcs_em, dst_em)
    x_elem1 = _tc_combine(agg_e1, cnt_e, x_elem0, W1_me_l, b1_me_l, W1_me_r,
                          lrelu=True)
    x_mat1 = _tc_combine(agg_m1, cnt_m, x_mat0, W1_em_l, b1_em_l, W1_em_r,
                         lrelu=True)

    agg_m2 = _sc_segsum(x_elem1.reshape(2 * N, HH), srcs_em, dst_em)
    x_mat2 = _tc_combine(agg_m2, cnt_m, x_mat1, W2_em_l, b2_em_l, W2_em_r,
                         lrelu=True)

    return _tc_head(x_mat2, W_out, b_out)


# speedup vs baseline: 1.5920x; 1.0078x over previous
import functools

import jax
import jax.numpy as jnp
from jax import lax
from jax.experimental import pallas as pl
from jax.experimental.pallas import tpu as pltpu
from jax.experimental.pallas import tpu_sc as plsc

N = 10000
E = 160000
H = 256
HH = 128
NTILE = 16
BATCH = 64
NB = 160
EPAD = NTILE * NB * BATCH
DUMP = N
NPAD = 10112
ROWS = NPAD
TROWS = NPAD // NTILE


def _dot_t(a, w):
    return lax.dot_general(a, w, (((1,), (1,)), ((), ())),
                           preferred_element_type=jnp.float32)


_BM = 2000


def _encoder_body(x_ref, w_ref, b_ref, emb_ref, out_ref):
    acc = _dot_t(x_ref[...], w_ref[...]) + b_ref[0] + emb_ref[...]
    out_ref[...] = acc[None]


def _tc_encoder(x_material, W_lin, b_lin, emb_material):
    b2 = b_lin.reshape(2, 1, HH)
    return pl.pallas_call(
        _encoder_body,
        grid=(2, N // _BM),
        in_specs=[
            pl.BlockSpec((_BM, H), lambda c, r: (r, 0)),
            pl.BlockSpec((HH, H), lambda c, r: (c, 0)),
            pl.BlockSpec((1, 1, HH), lambda c, r: (c, 0, 0)),
            pl.BlockSpec((_BM, HH), lambda c, r: (r, c)),
        ],
        out_specs=pl.BlockSpec((1, _BM, HH), lambda c, r: (c, r, 0)),
        out_shape=jax.ShapeDtypeStruct((2, N, HH), jnp.float32),
    )(x_material, W_lin, b2, emb_material)


def _combine_body(aggr_ref, cnt_ref, xd_ref, wl_ref, wr_ref, b_ref, out_ref,
                  *, lrelu):
    inv = 1.0 / jnp.maximum(cnt_ref[:, 0:1], 1.0)
    a0 = aggr_ref[0] * inv
    a1 = aggr_ref[1] * inv
    acc = (_dot_t(a0, wl_ref[:, 0:HH]) + _dot_t(a1, wl_ref[:, HH:H])
           + _dot_t(xd_ref[0], wr_ref[:, 0:HH]) + _dot_t(xd_ref[1], wr_ref[:, HH:H])
           + b_ref[0])
    if lrelu:
        acc = jnp.where(acc > 0, acc, 0.01 * acc)
    out_ref[...] = acc[None]


def _tc_combine(aggr, cnt, x_dst, W_l, b_l, W_r, lrelu):
    b2 = b_l.reshape(2, 1, HH)
    return pl.pallas_call(
        functools.partial(_combine_body, lrelu=lrelu),
        grid=(2, N // _BM),
        in_specs=[
            pl.BlockSpec((2, _BM, HH), lambda c, r: (0, r, 0)),
            pl.BlockSpec((_BM, HH), lambda c, r: (r, 0)),
            pl.BlockSpec((2, _BM, HH), lambda c, r: (0, r, 0)),
            pl.BlockSpec((HH, H), lambda c, r: (c, 0)),
            pl.BlockSpec((HH, H), lambda c, r: (c, 0)),
            pl.BlockSpec((1, 1, HH), lambda c, r: (c, 0, 0)),
        ],
        out_specs=pl.BlockSpec((1, _BM, HH), lambda c, r: (c, r, 0)),
        out_shape=jax.ShapeDtypeStruct((2, N, HH), jnp.float32),
    )(aggr, cnt, x_dst, W_l, W_r, b2)


def _head_body(x_ref, w_ref, b_ref, out_ref):
    out_ref[...] = (_dot_t(x_ref[0], w_ref[:, 0:HH])
                    + _dot_t(x_ref[1], w_ref[:, HH:H]) + b_ref[...])


def _tc_head(x, W_out, b_out):
    nout = W_out.shape[0]
    return pl.pallas_call(
        _head_body,
        grid=(N // _BM,),
        in_specs=[
            pl.BlockSpec((2, _BM, HH), lambda r: (0, r, 0)),
            pl.BlockSpec((nout, H), lambda r: (0, 0)),
            pl.BlockSpec((1, nout), lambda r: (0, 0)),
        ],
        out_specs=pl.BlockSpec((_BM, nout), lambda r: (r, 0)),
        out_shape=jax.ShapeDtypeStruct((N, nout), jnp.float32),
    )(x, W_out, b_out.reshape(1, nout))


_MESH = plsc.VectorSubcoreMesh(core_axis_name="c", subcore_axis_name="s")


def _fill(ref, rows, value):
    vec = jnp.full((16,), value, jnp.float32)

    def body(i, _):
        for j in range(ref.shape[1] // 16):
            ref[i, pl.ds(j * 16, 16)] = vec
        return 0

    lax.fori_loop(0, rows, body, 0)


def _zero_slab(zsrc, acc_sh, slab):
    nfull = TROWS // BATCH
    for k in range(nfull):
        pltpu.sync_copy(zsrc, acc_sh.at[pl.ds(slab + k * BATCH, BATCH)])
    rem = TROWS - nfull * BATCH
    if rem:
        pltpu.sync_copy(zsrc.at[pl.ds(0, rem)],
                        acc_sh.at[pl.ds(slab + nfull * BATCH, rem)])


def _make_segsum(W):
  @functools.partial(
    pl.kernel,
    out_type=jax.ShapeDtypeStruct((2, NPAD, W), jnp.float32),
    mesh=_MESH,
    scratch_types=[
        pltpu.VMEM((NB // 2, BATCH), jnp.int32),
        pltpu.VMEM((NB // 2, BATCH), jnp.int32),
        pltpu.VMEM((2, BATCH, W), jnp.float32),
        pltpu.SemaphoreType.DMA,
        pltpu.SemaphoreType.DMA,
        pltpu.VMEM_SHARED((ROWS, W), jnp.float32),
    ],
  )
  def _sc_segsum(x_hbm, srcs_hbm, dsts_hbm, out_hbm,
                 src_v, dst_v, rows_v, sem0, sem1, acc_sh):
      c = lax.axis_index("c")
      s = lax.axis_index("s")
      PH = NB // 2
      bufs = (rows_v.at[0], rows_v.at[1])
      sems = (sem0, sem1)
      _fill(bufs[0], BATCH, 0.0)
      slab = s * TROWS
      _zero_slab(bufs[0], acc_sh, slab)
      plsc.subcore_barrier()
      for p in range(2):
          pltpu.sync_copy(srcs_hbm.at[c, s, pl.ds(p * PH, PH)], src_v)
          pltpu.sync_copy(dsts_hbm.at[s, pl.ds(p * PH, PH)], dst_v)
          pltpu.async_copy(x_hbm.at[src_v.at[0]], bufs[0], sems[0])
          pltpu.async_copy(x_hbm.at[src_v.at[1]], bufs[1], sems[1])

          def body(i, _):
              for b in range(2):
                  j = 2 * i + b
                  pltpu.make_async_copy(x_hbm.at[src_v.at[j]], bufs[b],
                                        sems[b]).wait()
                  pltpu.sync_copy(bufs[b], acc_sh.at[dst_v.at[j]], add=True)

                  @pl.when(j + 2 < PH)
                  def _():
                      pltpu.async_copy(x_hbm.at[src_v.at[j + 2]], bufs[b],
                                       sems[b])
              return 0

          lax.fori_loop(0, PH // 2, body, 0)
      plsc.subcore_barrier()
      pltpu.sync_copy(acc_sh.at[pl.ds(slab, TROWS)],
                      out_hbm.at[c, pl.ds(slab, TROWS)])
  return _sc_segsum


_sc_segsum = _make_segsum(HH)


@functools.partial(
    pl.kernel,
    out_type=jax.ShapeDtypeStruct((2, NPAD, HH), jnp.float32),
    mesh=_MESH,
    scratch_types=[
        pltpu.VMEM((NB, BATCH), jnp.int32),
        pltpu.VMEM((BATCH, HH), jnp.float32),
        pltpu.VMEM_SHARED((ROWS, HH), jnp.float32),
    ],
)
def _sc_counts(dsts_hbm, out_hbm, dst_v, ones_v, acc_sh):
    c = lax.axis_index("c")
    s = lax.axis_index("s")
    pltpu.sync_copy(dsts_hbm.at[c, s], dst_v)
    _fill(ones_v, BATCH, 0.0)
    slab = s * TROWS
    _zero_slab(ones_v, acc_sh, slab)
    plsc.subcore_barrier()
    _fill(ones_v, BATCH, 1.0)

    def body(j, _):
        pltpu.sync_copy(ones_v, acc_sh.at[dst_v.at[j]], add=True)
        return 0

    lax.fori_loop(0, NB, body, 0)
    plsc.subcore_barrier()
    pltpu.sync_copy(acc_sh.at[pl.ds(slab, TROWS)],
                    out_hbm.at[c, pl.ds(slab, TROWS)])


def _prep_edges(ei):
    src = ei[0].astype(jnp.int32)
    dst = ei[1].astype(jnp.int32)
    pad = EPAD - E
    src = jnp.concatenate([src, jnp.zeros((pad,), jnp.int32)])
    dst = jnp.concatenate([dst, jnp.full((pad,), DUMP, jnp.int32)])
    src = src.reshape(NTILE, NB, BATCH)
    dst = dst.reshape(NTILE, NB, BATCH)
    srcs = jnp.stack([src, src + N])
    return srcs, dst


def kernel(x_material, emb_material, emb_element, W_lin, b_lin, W1_me_l,
           b1_me_l, W1_me_r, W1_em_l, b1_em_l, W1_em_r, W2_me_l, b2_me_l,
           W2_me_r, W2_em_l, b2_em_l, W2_em_r, W_out, b_out,
           node_id_material, node_id_element, ei_me, ei_em):
    srcs_me, dst_me = _prep_edges(ei_me)
    srcs_em, dst_em = _prep_edges(ei_em)

    cnts = _sc_counts(jnp.stack([dst_me, dst_em]))
    cnt_e = cnts[0]
    cnt_m = cnts[1]

    x_mat0 = _tc_encoder(x_material, W_lin, b_lin, emb_material)
    x_elem0 = emb_element.reshape(N, 2, HH).transpose(1, 0, 2)

    agg_e1 = _sc_segsum(x_mat0.reshape(2 * N, HH), srcs_me, dst_me)
    agg_m1 = _sc_segsum(x_elem0.reshape(2 * N, HH), srcs_em, dst_em)
    x_elem1 = _tc_combine(agg_e1, cnt_e, x_elem0, W1_me_l, b1_me_l, W1_me_r,
                          lrelu=True)
    x_mat1 = _tc_combine(agg_m1, cnt_m, x_mat0, W1_em_l, b1_em_l, W1_em_r,
                         lrelu=True)

    agg_m2 = _sc_segsum(x_elem1.reshape(2 * N, HH), srcs_em, dst_em)
    x_mat2 = _tc_combine(agg_m2, cnt_m, x_mat1, W2_em_l, b2_em_l, W2_em_r,
                         lrelu=True)

    return _tc_head(x_mat2, W_out, b_out)
